# baseline (speedup 1.0000x reference)
import functools

import jax
import jax.numpy as jnp
from jax import lax
from jax.experimental import pallas as pl


def _suppress_kernel(thresh_ref, x1_ref, y1_ref, x2_ref, y2_ref, area_ref,
                     sup_ref, *, n_real: int, rows: int, chunk_rows: int):
    n_chunks = rows // chunk_rows
    span = chunk_rows * 128
    lane2 = lax.broadcasted_iota(jnp.int32, (chunk_rows, 128), 1)
    row2 = lax.broadcasted_iota(jnp.int32, (chunk_rows, 128), 0)
    lane_f = lax.broadcasted_iota(jnp.int32, (rows, 128), 1)
    row_f = lax.broadcasted_iota(jnp.int32, (rows, 128), 0)
    pos_f = row_f * 128 + lane_f
    sup_ref[:, :] = jnp.where(pos_f >= n_real, 1.0, 0.0)
    thresh = thresh_ref[0, 0]
    lane1 = lax.broadcasted_iota(jnp.int32, (1, 128), 1)

    def do_update(i, x1_i, y1_i, x2_i, y2_i, area_i):
        def chunk_body(c, carry):
            r0 = c * chunk_rows
            x1c = x1_ref[pl.ds(r0, chunk_rows), :]
            y1c = y1_ref[pl.ds(r0, chunk_rows), :]
            x2c = x2_ref[pl.ds(r0, chunk_rows), :]
            y2c = y2_ref[pl.ds(r0, chunk_rows), :]
            areac = area_ref[pl.ds(r0, chunk_rows), :]
            xx1 = jnp.maximum(x1_i, x1c)
            yy1 = jnp.maximum(y1_i, y1c)
            xx2 = jnp.minimum(x2_i, x2c)
            yy2 = jnp.minimum(y2_i, y2c)
            w = jnp.maximum(xx2 - xx1, 0.0)
            h = jnp.maximum(yy2 - yy1, 0.0)
            inter = w * h
            iou = inter / (area_i + areac - inter)
            posc = (c * chunk_rows + row2) * 128 + lane2
            hit = (iou > thresh) & (posc > i)
            supc = sup_ref[pl.ds(r0, chunk_rows), :]
            sup_ref[pl.ds(r0, chunk_rows), :] = jnp.where(hit, 1.0, supc)
            return carry

        lax.fori_loop(i // span, n_chunks, chunk_body, 0)

    def row_body(r, carry):
        x1r = x1_ref[pl.ds(r, 1), :]
        y1r = y1_ref[pl.ds(r, 1), :]
        x2r = x2_ref[pl.ds(r, 1), :]
        y2r = y2_ref[pl.ds(r, 1), :]

        def next_kept(lmin):
            srow = sup_ref[pl.ds(r, 1), :]
            return jnp.min(jnp.where((srow < 0.5) & (lane1 >= lmin), lane1, 128))

        def wbody(l):
            onehot = lane1 == l
            x1_i = jnp.sum(jnp.where(onehot, x1r, 0.0))
            y1_i = jnp.sum(jnp.where(onehot, y1r, 0.0))
            x2_i = jnp.sum(jnp.where(onehot, x2r, 0.0))
            y2_i = jnp.sum(jnp.where(onehot, y2r, 0.0))
            area_i = (x2_i - x1_i) * (y2_i - y1_i)
            do_update(r * 128 + l, x1_i, y1_i, x2_i, y2_i, area_i)
            return next_kept(l + 1)

        lax.while_loop(lambda l: l < 128, wbody, next_kept(0))
        return carry

    lax.fori_loop(0, rows, row_body, 0)


def kernel(preds, thresh, max_proposals):
    n = preds.shape[0]
    npad = ((n + 1023) // 1024) * 1024
    rows = npad // 128
    chunk_rows = 32 if rows % 32 == 0 else rows

    scores = preds[:, 4]
    order = jnp.argsort(-scores)
    b = preds[order]
    coords = jnp.zeros((npad, 4), jnp.float32).at[:n].set(b[:, :4])
    x1 = coords[:, 0].reshape(rows, 128)
    y1 = coords[:, 1].reshape(rows, 128)
    x2 = coords[:, 2].reshape(rows, 128)
    y2 = coords[:, 3].reshape(rows, 128)
    areas = (x2 - x1) * (y2 - y1)
    thresh_arr = jnp.asarray(thresh, jnp.float32).reshape(1, 1)

    sup = pl.pallas_call(
        functools.partial(_suppress_kernel, n_real=n, rows=rows,
                          chunk_rows=chunk_rows),
        out_shape=jax.ShapeDtypeStruct((rows, 128), jnp.float32),
    )(thresh_arr, x1, y1, x2, y2, areas)

    supf = sup.reshape(-1)[:n]
    idx = jnp.arange(n, dtype=jnp.int32)
    keys = idx + supf.astype(jnp.int32) * n
    _, sel_pos = lax.top_k(-keys, 1000)
    keep1000 = order[sel_pos].astype(jnp.int32)
    sel = jnp.minimum(jnp.arange(1000), max_proposals - 1)
    keep = keep1000[sel]
    return preds[keep], keep


# speedup vs baseline: 1.0002x; 1.0002x over previous
import functools

import jax
import jax.numpy as jnp
from jax import lax
from jax.experimental import pallas as pl


def _suppress_kernel(thresh_ref, x1_ref, y1_ref, x2_ref, y2_ref, area_ref,
                     sup_ref, *, n_real: int, rows: int):
    q_rows = rows // 4
    lane_f = lax.broadcasted_iota(jnp.int32, (rows, 128), 1)
    row_f = lax.broadcasted_iota(jnp.int32, (rows, 128), 0)
    pos_f = row_f * 128 + lane_f
    sup_ref[:, :] = jnp.where(pos_f >= n_real, 1.0, 0.0)
    thresh = thresh_ref[0, 0]
    lane1 = lax.broadcasted_iota(jnp.int32, (1, 128), 1)
    lane_q = lax.broadcasted_iota(jnp.int32, (q_rows, 128), 1)
    row_q = lax.broadcasted_iota(jnp.int32, (q_rows, 128), 0)

    def row_body(r, carry):
        x1r = x1_ref[pl.ds(r, 1), :]
        y1r = y1_ref[pl.ds(r, 1), :]
        x2r = x2_ref[pl.ds(r, 1), :]
        y2r = y2_ref[pl.ds(r, 1), :]

        def lane_body(l, cc):
            srow = sup_ref[pl.ds(r, 1), :]
            onehot = lane1 == l
            s_i = jnp.sum(jnp.where(onehot, srow, 0.0))

            @pl.when(s_i == 0.0)
            def _():
                x1_i = jnp.sum(jnp.where(onehot, x1r, 0.0))
                y1_i = jnp.sum(jnp.where(onehot, y1r, 0.0))
                x2_i = jnp.sum(jnp.where(onehot, x2r, 0.0))
                y2_i = jnp.sum(jnp.where(onehot, y2r, 0.0))
                area_i = (x2_i - x1_i) * (y2_i - y1_i)
                i = r * 128 + l
                for q in range(4):
                    r0 = q * q_rows
                    q_end_pos = (q + 1) * q_rows * 128

                    @pl.when(i < q_end_pos - 1)
                    def _q(r0=r0, q=q):
                        x1c = x1_ref[r0:r0 + q_rows, :]
                        y1c = y1_ref[r0:r0 + q_rows, :]
                        x2c = x2_ref[r0:r0 + q_rows, :]
                        y2c = y2_ref[r0:r0 + q_rows, :]
                        areac = area_ref[r0:r0 + q_rows, :]
                        xx1 = jnp.maximum(x1_i, x1c)
                        yy1 = jnp.maximum(y1_i, y1c)
                        xx2 = jnp.minimum(x2_i, x2c)
                        yy2 = jnp.minimum(y2_i, y2c)
                        w = jnp.maximum(xx2 - xx1, 0.0)
                        h = jnp.maximum(yy2 - yy1, 0.0)
                        inter = w * h
                        iou = inter / (area_i + areac - inter)
                        posq = (r0 + row_q) * 128 + lane_q
                        hit = (iou > thresh) & (posq > i)
                        supq = sup_ref[r0:r0 + q_rows, :]
                        sup_ref[r0:r0 + q_rows, :] = jnp.where(hit, 1.0, supq)

            return cc

        srow0 = sup_ref[pl.ds(r, 1), :]

        @pl.when(jnp.min(srow0) < 0.5)
        def _():
            lax.fori_loop(0, 128, lane_body, 0)

        return carry

    lax.fori_loop(0, rows, row_body, 0)


def kernel(preds, thresh, max_proposals):
    n = preds.shape[0]
    npad = ((n + 1023) // 1024) * 1024
    rows = npad // 128

    scores = preds[:, 4]
    order = jnp.argsort(-scores)
    b = preds[order]
    coords = jnp.zeros((npad, 4), jnp.float32).at[:n].set(b[:, :4])
    x1 = coords[:, 0].reshape(rows, 128)
    y1 = coords[:, 1].reshape(rows, 128)
    x2 = coords[:, 2].reshape(rows, 128)
    y2 = coords[:, 3].reshape(rows, 128)
    areas = (x2 - x1) * (y2 - y1)
    thresh_arr = jnp.asarray(thresh, jnp.float32).reshape(1, 1)

    sup = pl.pallas_call(
        functools.partial(_suppress_kernel, n_real=n, rows=rows),
        out_shape=jax.ShapeDtypeStruct((rows, 128), jnp.float32),
    )(thresh_arr, x1, y1, x2, y2, areas)

    supf = sup.reshape(-1)[:n]
    idx = jnp.arange(n, dtype=jnp.int32)
    keys = idx + supf.astype(jnp.int32) * n
    _, sel_pos = lax.top_k(-keys, 1000)
    keep1000 = order[sel_pos].astype(jnp.int32)
    sel = jnp.minimum(jnp.arange(1000), max_proposals - 1)
    keep = keep1000[sel]
    return preds[keep], keep
